# baseline (speedup 1.0000x reference)
import functools

import jax
import jax.numpy as jnp
from jax import lax
from jax.experimental import pallas as pl
from jax.experimental.pallas import tpu as pltpu
from jax.experimental.pallas import tpu_sc as plsc

B, F, D, V = 4096, 26, 64, 100000
FP = 32
NC, NS = 2, 16
NW = NC * NS
BPW = B // NW
BB = 512


def _sc_gather_softmax(item_idx, table):
    mesh = plsc.VectorSubcoreMesh(
        core_axis_name="c", subcore_axis_name="s", num_cores=NC, num_subcores=NS
    )

    @functools.partial(
        pl.kernel,
        out_type=jax.ShapeDtypeStruct((B, FP), jnp.float32),
        mesh=mesh,
        scratch_types=[
            pltpu.VMEM((BPW,), jnp.int32),
            pltpu.VMEM((BPW, FP), jnp.float32),
            pltpu.SemaphoreType.DMA,
        ],
        compiler_params=pltpu.CompilerParams(
            needs_layout_passes=False, skip_device_barrier=True
        ),
    )
    def gather_kernel(idx_hbm, table_hbm, out_hbm, idx_s, rows_v, sem):
        wid = lax.axis_index("s") * NC + lax.axis_index("c")
        base = wid * BPW
        pltpu.sync_copy(idx_hbm.at[pl.ds(base, BPW)], idx_s)

        CH = 16

        def fire(c, _):
            j0 = c * CH
            vec = idx_s[pl.ds(j0, 16)]
            for u in range(CH):
                pltpu.async_copy(
                    table_hbm.at[vec[u]], rows_v.at[j0 + u, pl.ds(0, F)], sem
                )
            for u in range(CH):
                pltpu.make_async_copy(
                    table_hbm.at[0], rows_v.at[j0 + u, pl.ds(0, F)], sem
                ).wait()
            return ()

        lax.fori_loop(0, BPW // CH, fire, (), unroll=False)

        lanes = lax.iota(jnp.int32, 16)
        msk1 = lanes < (F - 16)

        def smax(j, _):
            x0 = rows_v[j, pl.ds(0, 16)]
            x1 = rows_v[j, pl.ds(16, 16)]
            x1m = jnp.where(msk1, x1, -jnp.inf)
            m = jnp.maximum(jnp.max(x0), jnp.max(x1m))
            e0 = jnp.exp(x0 - m)
            e1 = jnp.where(msk1, jnp.exp(x1 - m), 0.0)
            s = jnp.sum(e0) + jnp.sum(e1)
            sv = jnp.full((16,), s, jnp.float32)
            rows_v[j, pl.ds(0, 16)] = e0 / sv
            rows_v[j, pl.ds(16, 16)] = e1 / sv
            return ()

        lax.fori_loop(0, BPW, smax, (), unroll=False)
        pltpu.sync_copy(rows_v, out_hbm.at[pl.ds(base, BPW)])

    return gather_kernel(item_idx, table)


NQ = 8
SUB = BB // NQ


def _start_chunk(stack_hbm, bufs, sems, step, slot):
    for q in range(NQ):
        pltpu.make_async_copy(
            stack_hbm.at[pl.ds(step * BB + q * SUB, SUB)],
            bufs.at[slot, pl.ds(q * SUB, SUB)],
            sems.at[slot, q],
        ).start()


def _wait_chunk(stack_hbm, bufs, sems, step, slot):
    for q in range(NQ):
        pltpu.make_async_copy(
            stack_hbm.at[pl.ds(step * BB + q * SUB, SUB)],
            bufs.at[slot, pl.ds(q * SUB, SUB)],
            sems.at[slot, q],
        ).wait()


def _pool_body(w_ref, stack_hbm, out_ref, bufs, sems):
    i = pl.program_id(0)
    nsteps = B // BB

    @pl.when(i == 0)
    def _prologue():
        _start_chunk(stack_hbm, bufs, sems, 0, 0)

    @pl.when(i + 1 < nsteps)
    def _prefetch():
        _start_chunk(stack_hbm, bufs, sems, i + 1, (i + 1) % 2)

    slot = i % 2
    _wait_chunk(stack_hbm, bufs, sems, i, slot)
    x = bufs[slot]
    w = w_ref[...]
    out_ref[...] = jnp.sum(w[:, :F, None] * x, axis=1)


def _tc_pool(weights, stack_embedding):
    return pl.pallas_call(
        _pool_body,
        grid=(B // BB,),
        in_specs=[
            pl.BlockSpec((BB, FP), lambda i: (i, 0)),
            pl.BlockSpec(memory_space=pltpu.MemorySpace.HBM),
        ],
        out_specs=pl.BlockSpec((BB, D), lambda i: (i, 0)),
        out_shape=jax.ShapeDtypeStruct((B, D), jnp.float32),
        scratch_shapes=[
            pltpu.VMEM((2, BB, F, D), jnp.float32),
            pltpu.SemaphoreType.DMA((2, NQ)),
        ],
        compiler_params=pltpu.CompilerParams(skip_device_barrier=True),
    )(weights, stack_embedding)


def kernel(stack_embedding, item_input, alpha_embeddings):
    item_idx = jnp.reshape(item_input, (B,)).astype(jnp.int32)
    weights = _sc_gather_softmax(item_idx, alpha_embeddings)
    return _tc_pool(weights, stack_embedding)


# speedup vs baseline: 2.6399x; 2.6399x over previous
import functools

import jax
import jax.numpy as jnp
from jax import lax
from jax.experimental import pallas as pl
from jax.experimental.pallas import tpu as pltpu
from jax.experimental.pallas import tpu_sc as plsc

B, F, D, V = 4096, 26, 64, 100000
NC, NS = 2, 16
NW = NC * NS
BPW = B // NW
BB = 1024


def _sc_gather(item_idx, table_t):
    mesh = plsc.VectorSubcoreMesh(
        core_axis_name="c", subcore_axis_name="s", num_cores=NC, num_subcores=NS
    )

    @functools.partial(
        pl.kernel,
        out_type=jax.ShapeDtypeStruct((F, B), jnp.float32),
        mesh=mesh,
        scratch_types=[
            pltpu.VMEM((BPW,), jnp.int32),
            pltpu.VMEM((F, BPW), jnp.float32),
            pltpu.SemaphoreType.DMA,
        ],
        compiler_params=pltpu.CompilerParams(
            needs_layout_passes=False,
            skip_device_barrier=True,
            use_tc_tiling_on_sc=False,
        ),
    )
    def gather_kernel(idx_hbm, table_hbm, out_hbm, idx_v, rows_v, sem):
        wid = lax.axis_index("s") * NC + lax.axis_index("c")
        base = wid * BPW
        pltpu.sync_copy(idx_hbm.at[pl.ds(base, BPW)], idx_v)
        for f in range(F):
            pltpu.async_copy(
                table_hbm.at[f].at[idx_v], rows_v.at[f], sem
            )
        for f in range(F):
            pltpu.make_async_copy(
                table_hbm.at[f].at[idx_v], rows_v.at[f], sem
            ).wait()
        pltpu.sync_copy(rows_v, out_hbm.at[:, pl.ds(base, BPW)])

    return gather_kernel(item_idx, table_t)


def _pool_body(a_ref, x_ref, out_ref):
    a = a_ref[...]
    m = jnp.max(a, axis=0, keepdims=True)
    e = jnp.exp(a - m)
    w = e / jnp.sum(e, axis=0, keepdims=True)
    acc = x_ref[0] * w[0:1, :]
    for f in range(1, F):
        acc = acc + x_ref[f] * w[f : f + 1, :]
    out_ref[...] = acc


def _tc_pool(alpha_t, stack_t):
    return pl.pallas_call(
        _pool_body,
        grid=(B // BB,),
        in_specs=[
            pl.BlockSpec((F, BB), lambda i: (0, i)),
            pl.BlockSpec((F, D, BB), lambda i: (0, 0, i)),
        ],
        out_specs=pl.BlockSpec((D, BB), lambda i: (0, i)),
        out_shape=jax.ShapeDtypeStruct((D, B), jnp.float32),
        compiler_params=pltpu.CompilerParams(skip_device_barrier=True),
    )(alpha_t, stack_t)


def kernel(stack_embedding, item_input, alpha_embeddings):
    item_idx = jnp.reshape(item_input, (B,)).astype(jnp.int32)
    table_t = jnp.transpose(alpha_embeddings)
    stack_t = jnp.transpose(stack_embedding, (1, 2, 0))
    alpha_t = _sc_gather(item_idx, table_t)
    out_t = _tc_pool(alpha_t, stack_t)
    return jnp.transpose(out_t)
